# baseline (speedup 1.0000x reference)
import functools
import jax
import jax.numpy as jnp
from jax import lax
from jax.experimental import pallas as pl
from jax.experimental.pallas import tpu as pltpu
from jax.experimental.pallas import tpu_sc as plsc

N_NODES = 10000
CH = 128
NC, NS = 2, 16
NW = NC * NS
EB = 128
N_PAD = 10240
ROW_BLK = 2048
GRID = N_PAD // ROW_BLK


def _sc_body(*refs, blocks_per_w, with_cnt):
    if with_cnt:
        (y_hbm, src_hbm, dst_hbm, zacc_hbm, zcnt_hbm, acc_out, cnt_out,
         sv0, sv1, dv0, dv1, rows_v, ones_v, acc_sh, cnt_sh,
         sem, si0, si1) = refs
    else:
        (y_hbm, src_hbm, dst_hbm, zacc_hbm, acc_out,
         sv0, sv1, dv0, dv1, rows_v, acc_sh,
         sem, si0, si1) = refs
    src_v = [sv0, sv1]
    dst_v = [dv0, dv1]
    si = [si0, si1]
    c = lax.axis_index("c")
    s = lax.axis_index("s")
    wid = c * NS + s

    zrows = N_PAD // NS
    pltpu.sync_copy(zacc_hbm.at[pl.ds(s * zrows, zrows)],
                    acc_sh.at[pl.ds(s * zrows, zrows)])
    if with_cnt:
        pltpu.sync_copy(zcnt_hbm.at[pl.ds(s * zrows, zrows)],
                        cnt_sh.at[pl.ds(s * zrows, zrows)])
        for i in range(EB // 16):
            ones_v[pl.ds(i * 16, 16)] = jnp.ones((16,), jnp.float32)
    plsc.subcore_barrier()

    def idx_fetch(j, p):
        base = pl.multiple_of((wid * blocks_per_w + j) * EB, EB)
        pltpu.async_copy(src_hbm.at[pl.ds(base, EB)], src_v[p], si[p])
        pltpu.async_copy(dst_hbm.at[pl.ds(base, EB)], dst_v[p], si[p])

    def idx_wait(p):
        base = pl.multiple_of(0, EB)
        pltpu.make_async_copy(src_hbm.at[pl.ds(base, EB)],
                              src_v[p], si[p]).wait()
        pltpu.make_async_copy(dst_hbm.at[pl.ds(base, EB)],
                              dst_v[p], si[p]).wait()

    idx_fetch(0, 0)

    def block(j, p):
        idx_wait(p)

        @pl.when(j + 1 < blocks_per_w)
        def _():
            idx_fetch(j + 1, 1 - p)
        pltpu.async_copy(y_hbm.at[src_v[p]], rows_v, sem).wait()
        pltpu.sync_copy(rows_v, acc_sh.at[dst_v[p]], add=True)
        if with_cnt:
            pltpu.sync_copy(ones_v, cnt_sh.at[dst_v[p]], add=True)

    def pairbody(i, carry):
        block(2 * i, 0)
        block(2 * i + 1, 1)
        return carry

    lax.fori_loop(0, blocks_per_w // 2, pairbody, 0)
    plsc.subcore_barrier()

    pltpu.sync_copy(acc_sh.at[pl.ds(s * zrows, zrows)],
                    acc_out.at[c, pl.ds(s * zrows, zrows)])
    if with_cnt:
        pltpu.sync_copy(cnt_sh.at[pl.ds(s * zrows, zrows)],
                        cnt_out.at[c, pl.ds(s * zrows, zrows)])


def _make_sc_agg(blocks_per_w, with_cnt):
    mesh = plsc.VectorSubcoreMesh(core_axis_name="c", subcore_axis_name="s")
    out_type = [jax.ShapeDtypeStruct((NC, N_PAD, CH), jnp.float32)]
    scratch = [
        pltpu.VMEM((EB,), jnp.int32),
        pltpu.VMEM((EB,), jnp.int32),
        pltpu.VMEM((EB,), jnp.int32),
        pltpu.VMEM((EB,), jnp.int32),
        pltpu.VMEM((EB, CH), jnp.float32),
    ]
    if with_cnt:
        out_type.append(jax.ShapeDtypeStruct((NC, N_PAD), jnp.float32))
        scratch.append(pltpu.VMEM((EB,), jnp.float32))
    scratch.append(pltpu.VMEM_SHARED((N_PAD, CH), jnp.float32))
    if with_cnt:
        scratch.append(pltpu.VMEM_SHARED((N_PAD,), jnp.float32))
    scratch += [pltpu.SemaphoreType.DMA for _ in range(3)]
    return pl.kernel(
        functools.partial(_sc_body, blocks_per_w=blocks_per_w,
                          with_cnt=with_cnt),
        out_type=out_type,
        mesh=mesh,
        scratch_types=scratch,
        name="sc_segment_sum_cnt" if with_cnt else "sc_segment_sum",
    )


def _dotT(a, w):
    return lax.dot_general(a, w, (((1,), (1,)), ((), ())),
                           preferred_element_type=jnp.float32)


def _tc_first_body(x_ref, wl_ref, wr_ref, bl_ref, y_ref, r_ref):
    x = x_ref[...]
    y_ref[...] = _dotT(x, wl_ref[...])
    r_ref[...] = _dotT(x, wr_ref[...]) + bl_ref[...]


def _tc_mid_body(acc_ref, cnt_ref, rp_ref, wl_ref, wr_ref, bl_ref,
                 y_ref, r_ref):
    a = acc_ref[0] + acc_ref[1]
    cnt = cnt_ref[0] + cnt_ref[1]
    recip = 1.0 / jnp.maximum(cnt, 1.0)
    h = jnp.maximum(a * recip + rp_ref[...], 0.0)
    y_ref[...] = _dotT(h, wl_ref[...])
    r_ref[...] = _dotT(h, wr_ref[...]) + bl_ref[...]


def _tc_last_body(acc_ref, cnt_ref, rp_ref, out_ref):
    a = acc_ref[0] + acc_ref[1]
    cnt = cnt_ref[0] + cnt_ref[1]
    recip = 1.0 / jnp.maximum(cnt, 1.0)
    out_ref[...] = jax.nn.sigmoid(a * recip + rp_ref[...])


_row_spec = pl.BlockSpec((ROW_BLK, CH), lambda i: (i, 0))
_acc_spec = pl.BlockSpec((NC, ROW_BLK, CH), lambda i: (0, i, 0))
_cnt_spec = pl.BlockSpec((NC, ROW_BLK, 1), lambda i: (0, i, 0))
_w_spec = pl.BlockSpec((CH, CH), lambda i: (0, 0))
_b_spec = pl.BlockSpec((1, CH), lambda i: (0, 0))
_f32 = lambda shape: jax.ShapeDtypeStruct(shape, jnp.float32)

_tc_first = pl.pallas_call(
    _tc_first_body, grid=(GRID,),
    in_specs=[_row_spec, _w_spec, _w_spec, _b_spec],
    out_specs=[_row_spec, _row_spec],
    out_shape=[_f32((N_PAD, CH)), _f32((N_PAD, CH))],
)

_tc_mid = pl.pallas_call(
    _tc_mid_body, grid=(GRID,),
    in_specs=[_acc_spec, _cnt_spec, _row_spec, _w_spec, _w_spec, _b_spec],
    out_specs=[_row_spec, _row_spec],
    out_shape=[_f32((N_PAD, CH)), _f32((N_PAD, CH))],
)

OUT_BLK = 2000

_tc_last = pl.pallas_call(
    _tc_last_body, grid=(N_NODES // OUT_BLK,),
    in_specs=[pl.BlockSpec((NC, OUT_BLK, CH), lambda i: (0, i, 0)),
              pl.BlockSpec((NC, OUT_BLK, 1), lambda i: (0, i, 0)),
              pl.BlockSpec((OUT_BLK, CH), lambda i: (i, 0))],
    out_specs=pl.BlockSpec((OUT_BLK, CH), lambda i: (i, 0)),
    out_shape=_f32((N_NODES, CH)),
)


def kernel(x, edge_index, Wl0, bl0, Wr0, Wl1, bl1, Wr1, Wl2, bl2, Wr2):
    src = edge_index[0].astype(jnp.int32)
    dst = edge_index[1].astype(jnp.int32)
    n_edges = src.shape[0]
    bpw = -(-n_edges // (NW * EB))
    bpw = -(-bpw // 2) * 2
    e_pad = NW * bpw * EB
    src_p = jnp.concatenate(
        [src, jnp.zeros((e_pad - n_edges,), jnp.int32)])
    dst_p = jnp.concatenate(
        [dst, jnp.full((e_pad - n_edges,), N_NODES, jnp.int32)])
    x_p = jnp.pad(x, ((0, N_PAD - N_NODES), (0, 0)))
    zacc = jnp.zeros((N_PAD, CH), jnp.float32)
    zcnt = jnp.zeros((N_PAD,), jnp.float32)

    sc_agg_cnt = _make_sc_agg(bpw, True)
    sc_agg = _make_sc_agg(bpw, False)

    y0, r0 = _tc_first(x_p, Wl0, Wr0, bl0.reshape(1, CH))
    acc0, cnt = sc_agg_cnt(y0, src_p, dst_p, zacc, zcnt)
    cnt3 = cnt.reshape(NC, N_PAD, 1)
    y1, r1 = _tc_mid(acc0, cnt3, r0, Wl1, Wr1, bl1.reshape(1, CH))
    (acc1,) = sc_agg(y1, src_p, dst_p, zacc)
    y2, r2 = _tc_mid(acc1, cnt3, r1, Wl2, Wr2, bl2.reshape(1, CH))
    (acc2,) = sc_agg(y2, src_p, dst_p, zacc)
    return _tc_last(acc2, cnt3, r2)


# speedup vs baseline: 1.3510x; 1.3510x over previous
import functools
import jax
import jax.numpy as jnp
from jax import lax
from jax.experimental import pallas as pl
from jax.experimental.pallas import tpu as pltpu
from jax.experimental.pallas import tpu_sc as plsc

N_NODES = 10000
CH = 128
NC, NS = 2, 16
NW = NC * NS
EB = 128
N_PAD = 10240
ROW_BLK = 2048
GRID = N_PAD // ROW_BLK


def _sc_body(*refs, blocks_per_w, with_cnt):
    if with_cnt:
        (y_hbm, src_hbm, dst_hbm, zacc_hbm, zcnt_hbm, acc_out, cnt_out,
         src_v, dst_v, rows_v, ones_v, acc_sh, cnt_sh, sem) = refs
    else:
        (y_hbm, src_hbm, dst_hbm, zacc_hbm, acc_out,
         src_v, dst_v, rows_v, acc_sh, sem) = refs
    c = lax.axis_index("c")
    s = lax.axis_index("s")
    wid = c * NS + s

    zrows = N_PAD // NS
    pltpu.sync_copy(zacc_hbm.at[pl.ds(s * zrows, zrows)],
                    acc_sh.at[pl.ds(s * zrows, zrows)])
    if with_cnt:
        pltpu.sync_copy(zcnt_hbm.at[pl.ds(s * zrows, zrows)],
                        cnt_sh.at[pl.ds(s * zrows, zrows)])
        for i in range(EB // 16):
            ones_v[pl.ds(i * 16, 16)] = jnp.ones((16,), jnp.float32)
    plsc.subcore_barrier()

    def body(j, carry):
        base = pl.multiple_of((wid * blocks_per_w + j) * EB, EB)
        pltpu.sync_copy(src_hbm.at[pl.ds(base, EB)], src_v)
        pltpu.sync_copy(dst_hbm.at[pl.ds(base, EB)], dst_v)
        pltpu.async_copy(y_hbm.at[src_v], rows_v, sem).wait()
        pltpu.sync_copy(rows_v, acc_sh.at[dst_v], add=True)
        if with_cnt:
            pltpu.sync_copy(ones_v, cnt_sh.at[dst_v], add=True)
        return carry

    lax.fori_loop(0, blocks_per_w, body, 0)
    plsc.subcore_barrier()

    pltpu.sync_copy(acc_sh.at[pl.ds(s * zrows, zrows)],
                    acc_out.at[c, pl.ds(s * zrows, zrows)])
    if with_cnt:
        pltpu.sync_copy(cnt_sh.at[pl.ds(s * zrows, zrows)],
                        cnt_out.at[c, pl.ds(s * zrows, zrows)])


def _make_sc_agg(blocks_per_w, with_cnt):
    mesh = plsc.VectorSubcoreMesh(core_axis_name="c", subcore_axis_name="s")
    out_type = [jax.ShapeDtypeStruct((NC, N_PAD, CH), jnp.float32)]
    scratch = [
        pltpu.VMEM((EB,), jnp.int32),
        pltpu.VMEM((EB,), jnp.int32),
        pltpu.VMEM((EB, CH), jnp.float32),
    ]
    if with_cnt:
        out_type.append(jax.ShapeDtypeStruct((NC, N_PAD), jnp.float32))
        scratch.append(pltpu.VMEM((EB,), jnp.float32))
    scratch.append(pltpu.VMEM_SHARED((N_PAD, CH), jnp.float32))
    if with_cnt:
        scratch.append(pltpu.VMEM_SHARED((N_PAD,), jnp.float32))
    scratch.append(pltpu.SemaphoreType.DMA)
    return pl.kernel(
        functools.partial(_sc_body, blocks_per_w=blocks_per_w,
                          with_cnt=with_cnt),
        out_type=out_type,
        mesh=mesh,
        scratch_types=scratch,
        name="sc_segment_sum_cnt" if with_cnt else "sc_segment_sum",
    )


def _dotT(a, w):
    return lax.dot_general(a, w, (((1,), (1,)), ((), ())),
                           preferred_element_type=jnp.float32)


def _tc_first_body(x_ref, wl_ref, wr_ref, bl_ref, y_ref, r_ref):
    x = x_ref[...]
    y_ref[...] = _dotT(x, wl_ref[...])
    r_ref[...] = _dotT(x, wr_ref[...]) + bl_ref[...]


def _tc_mid_body(acc_ref, cnt_ref, rp_ref, wl_ref, wr_ref, bl_ref,
                 y_ref, r_ref):
    a = acc_ref[0] + acc_ref[1]
    cnt = cnt_ref[0] + cnt_ref[1]
    recip = 1.0 / jnp.maximum(cnt, 1.0)
    h = jnp.maximum(a * recip + rp_ref[...], 0.0)
    y_ref[...] = _dotT(h, wl_ref[...])
    r_ref[...] = _dotT(h, wr_ref[...]) + bl_ref[...]


def _tc_last_body(acc_ref, cnt_ref, rp_ref, out_ref):
    a = acc_ref[0] + acc_ref[1]
    cnt = cnt_ref[0] + cnt_ref[1]
    recip = 1.0 / jnp.maximum(cnt, 1.0)
    out_ref[...] = jax.nn.sigmoid(a * recip + rp_ref[...])


_row_spec = pl.BlockSpec((ROW_BLK, CH), lambda i: (i, 0))
_acc_spec = pl.BlockSpec((NC, ROW_BLK, CH), lambda i: (0, i, 0))
_cnt_spec = pl.BlockSpec((NC, ROW_BLK, 1), lambda i: (0, i, 0))
_w_spec = pl.BlockSpec((CH, CH), lambda i: (0, 0))
_b_spec = pl.BlockSpec((1, CH), lambda i: (0, 0))
_f32 = lambda shape: jax.ShapeDtypeStruct(shape, jnp.float32)

_tc_first = pl.pallas_call(
    _tc_first_body, grid=(GRID,),
    in_specs=[_row_spec, _w_spec, _w_spec, _b_spec],
    out_specs=[_row_spec, _row_spec],
    out_shape=[_f32((N_PAD, CH)), _f32((N_PAD, CH))],
)

_tc_mid = pl.pallas_call(
    _tc_mid_body, grid=(GRID,),
    in_specs=[_acc_spec, _cnt_spec, _row_spec, _w_spec, _w_spec, _b_spec],
    out_specs=[_row_spec, _row_spec],
    out_shape=[_f32((N_PAD, CH)), _f32((N_PAD, CH))],
)

OUT_BLK = 2000

_tc_last = pl.pallas_call(
    _tc_last_body, grid=(N_NODES // OUT_BLK,),
    in_specs=[pl.BlockSpec((NC, OUT_BLK, CH), lambda i: (0, i, 0)),
              pl.BlockSpec((NC, OUT_BLK, 1), lambda i: (0, i, 0)),
              pl.BlockSpec((OUT_BLK, CH), lambda i: (i, 0))],
    out_specs=pl.BlockSpec((OUT_BLK, CH), lambda i: (i, 0)),
    out_shape=_f32((N_NODES, CH)),
)


def kernel(x, edge_index, Wl0, bl0, Wr0, Wl1, bl1, Wr1, Wl2, bl2, Wr2):
    src = edge_index[0].astype(jnp.int32)
    dst = edge_index[1].astype(jnp.int32)
    n_edges = src.shape[0]
    bpw = -(-n_edges // (NW * EB))
    e_pad = NW * bpw * EB
    src_p = jnp.concatenate(
        [src, jnp.zeros((e_pad - n_edges,), jnp.int32)])
    dst_p = jnp.concatenate(
        [dst, jnp.full((e_pad - n_edges,), N_NODES, jnp.int32)])
    x_p = jnp.pad(x, ((0, N_PAD - N_NODES), (0, 0)))
    zacc = jnp.zeros((N_PAD, CH), jnp.float32)
    zcnt = jnp.zeros((N_PAD,), jnp.float32)

    sc_agg_cnt = _make_sc_agg(bpw, True)
    sc_agg = _make_sc_agg(bpw, False)

    y0, r0 = _tc_first(x_p, Wl0, Wr0, bl0.reshape(1, CH))
    acc0, cnt = sc_agg_cnt(y0, src_p, dst_p, zacc, zcnt)
    cnt3 = cnt.reshape(NC, N_PAD, 1)
    y1, r1 = _tc_mid(acc0, cnt3, r0, Wl1, Wr1, bl1.reshape(1, CH))
    (acc1,) = sc_agg(y1, src_p, dst_p, zacc)
    y2, r2 = _tc_mid(acc1, cnt3, r1, Wl2, Wr2, bl2.reshape(1, CH))
    (acc2,) = sc_agg(y2, src_p, dst_p, zacc)
    return _tc_last(acc2, cnt3, r2)
